# Initial kernel scaffold; baseline (speedup 1.0000x reference)
import jax
import jax.numpy as jnp
from jax.experimental import pallas as pl


def kernel(index, targets, token_embedding_table):
    raise NotImplementedError("write your pallas kernel here")


# speedup vs baseline: 2.0552x; 2.0552x over previous
import functools

import jax
import jax.numpy as jnp
from jax import lax
from jax.experimental import pallas as pl
from jax.experimental.pallas import tpu as pltpu
from jax.experimental.pallas import tpu_sc as plsc

VOCAB = 1000
VPAD = 1024
BT = 51200
NC, NS = 2, 16
NW = NC * NS
PER_TILE = BT // NW
CHUNK = 64
NCHUNK = PER_TILE // CHUNK


def _row_logsumexp(table):
    def body(t_ref, o_ref):
        x = t_ref[...]
        m = jnp.max(x, axis=1, keepdims=True)
        s = jnp.sum(jnp.exp(x - m), axis=1, keepdims=True)
        o_ref[...] = jnp.log(s) + m

    return pl.pallas_call(
        body,
        out_shape=jax.ShapeDtypeStruct((VOCAB, 1), jnp.float32),
    )(table)


def _sc_gather_and_loss(idx_flat, tgt_flat, table, table_flat, lse):
    mesh = plsc.VectorSubcoreMesh(core_axis_name="c", subcore_axis_name="s")

    @functools.partial(
        pl.kernel,
        mesh=mesh,
        out_type=[
            jax.ShapeDtypeStruct((BT, VPAD), jnp.float32),
            jax.ShapeDtypeStruct((NW, 16), jnp.float32),
        ],
        scratch_types=[
            pltpu.VMEM((PER_TILE,), jnp.int32),
            pltpu.VMEM((PER_TILE,), jnp.int32),
            pltpu.VMEM((CHUNK, VPAD), jnp.float32),
            pltpu.VMEM((PER_TILE,), jnp.float32),
            pltpu.VMEM((PER_TILE,), jnp.float32),
            pltpu.VMEM((16,), jnp.float32),
            pltpu.SemaphoreType.DMA,
            pltpu.SemaphoreType.DMA,
        ],
    )
    def k(idx_hbm, tgt_hbm, table_hbm, tflat_hbm, lse_hbm, out_hbm, part_hbm,
          idx_v, tgt_v, rows_v, tl_v, ls_v, acc_v, sem_r, sem_e):
        wid = lax.axis_index("s") * NC + lax.axis_index("c")
        base = wid * PER_TILE
        pltpu.sync_copy(idx_hbm.at[pl.ds(base, PER_TILE)], idx_v)
        pltpu.sync_copy(tgt_hbm.at[pl.ds(base, PER_TILE)], tgt_v)

        def chunk_body(c, acc):
            o = c * CHUNK
            row_cp = pltpu.async_copy(
                table_hbm.at[idx_v.at[pl.ds(o, CHUNK)]], rows_v, sem_r
            )
            elem_cps = []
            for g in range(CHUNK // 16):
                og = o + g * 16
                idx16 = idx_v[pl.ds(og, 16)]
                t16 = tgt_v[pl.ds(og, 16)]
                flat16 = idx16 * VOCAB + t16
                elem_cps.append(pltpu.async_copy(
                    tflat_hbm.at[flat16], tl_v.at[pl.ds(og, 16)], sem_e
                ))
                elem_cps.append(pltpu.async_copy(
                    lse_hbm.at[idx16], ls_v.at[pl.ds(og, 16)], sem_e
                ))
            row_cp.wait()
            pltpu.sync_copy(rows_v, out_hbm.at[pl.ds(base + o, CHUNK)])
            for cp in elem_cps:
                cp.wait()
            return acc

        lax.fori_loop(0, NCHUNK, chunk_body, 0, unroll=False)

        def loss_body(g, acc):
            og = g * 16
            return acc + (ls_v[pl.ds(og, 16)] - tl_v[pl.ds(og, 16)])

        acc = lax.fori_loop(
            0, PER_TILE // 16, loss_body, jnp.zeros((16,), jnp.float32)
        )
        acc_v[...] = acc * (1.0 / BT)
        pltpu.sync_copy(acc_v, part_hbm.at[wid])

    return k(idx_flat, tgt_flat, table, table_flat, lse)


def kernel(index, targets, token_embedding_table):
    idx_flat = jnp.transpose(index).reshape(-1)
    tgt_flat = targets.reshape(-1)
    lse = _row_logsumexp(token_embedding_table).reshape(VOCAB)
    table_pad = jnp.pad(token_embedding_table, ((0, 0), (0, VPAD - VOCAB)))
    logits_pad, part = _sc_gather_and_loss(
        idx_flat, tgt_flat, table_pad,
        token_embedding_table.reshape(-1), lse
    )
    loss = jnp.sum(part)
    return (logits_pad[:, :VOCAB], loss)


# speedup vs baseline: 2.0861x; 1.0151x over previous
import functools

import jax
import jax.numpy as jnp
from jax import lax
from jax.experimental import pallas as pl
from jax.experimental.pallas import tpu as pltpu
from jax.experimental.pallas import tpu_sc as plsc

VOCAB = 1000
VPAD = 1024
BT = 51200
NC, NS = 2, 16
NW = NC * NS
PER_TILE = BT // NW
CHUNK = 32
NCHUNK = PER_TILE // CHUNK


def _row_logsumexp(table):
    def body(t_ref, o_ref):
        x = t_ref[...]
        m = jnp.max(x, axis=1, keepdims=True)
        s = jnp.sum(jnp.exp(x - m), axis=1, keepdims=True)
        o_ref[...] = jnp.log(s) + m

    return pl.pallas_call(
        body,
        out_shape=jax.ShapeDtypeStruct((VOCAB, 1), jnp.float32),
    )(table)


def _sc_gather_and_loss(idx_flat, tgt_flat, table_pad, table_flat, lse):
    mesh = plsc.VectorSubcoreMesh(core_axis_name="c", subcore_axis_name="s")

    @functools.partial(
        pl.kernel,
        mesh=mesh,
        out_type=[
            jax.ShapeDtypeStruct((BT, VPAD), jnp.float32),
            jax.ShapeDtypeStruct((NW, 16), jnp.float32),
        ],
        scratch_types=[
            pltpu.VMEM((PER_TILE,), jnp.int32),
            pltpu.VMEM((PER_TILE,), jnp.int32),
            pltpu.VMEM((CHUNK, VPAD), jnp.float32),
            pltpu.VMEM((CHUNK, VPAD), jnp.float32),
            pltpu.VMEM((PER_TILE,), jnp.float32),
            pltpu.VMEM((PER_TILE,), jnp.float32),
            pltpu.VMEM((16,), jnp.float32),
            pltpu.SemaphoreType.DMA,
            pltpu.SemaphoreType.DMA,
            pltpu.SemaphoreType.DMA,
            pltpu.SemaphoreType.DMA,
            pltpu.SemaphoreType.DMA,
        ],
    )
    def k(idx_hbm, tgt_hbm, table_hbm, tflat_hbm, lse_hbm, out_hbm, part_hbm,
          idx_v, tgt_v, rows0_v, rows1_v, tl_v, ls_v, acc_v,
          sem_g0, sem_g1, sem_s0, sem_s1, sem_e):
        rows = (rows0_v, rows1_v)
        sem_g = (sem_g0, sem_g1)
        sem_s = (sem_s0, sem_s1)
        wid = lax.axis_index("s") * NC + lax.axis_index("c")
        base = wid * PER_TILE
        pltpu.sync_copy(idx_hbm.at[pl.ds(base, PER_TILE)], idx_v)
        pltpu.sync_copy(tgt_hbm.at[pl.ds(base, PER_TILE)], tgt_v)

        def gather_rows(c, b):
            return pltpu.async_copy(
                table_hbm.at[idx_v.at[pl.ds(c * CHUNK, CHUNK)]],
                rows[b], sem_g[b],
            )

        gather_rows(0, 0)
        gather_rows(1, 1)

        def pair_body(p, acc):
            for b in range(2):
                c = 2 * p + b
                o = c * CHUNK
                elem_cps = []
                for g in range(CHUNK // 16):
                    og = o + g * 16
                    idx16 = idx_v[pl.ds(og, 16)]
                    t16 = tgt_v[pl.ds(og, 16)]
                    flat16 = idx16 * VOCAB + t16
                    elem_cps.append(pltpu.async_copy(
                        tflat_hbm.at[flat16], tl_v.at[pl.ds(og, 16)], sem_e
                    ))
                    elem_cps.append(pltpu.async_copy(
                        lse_hbm.at[idx16], ls_v.at[pl.ds(og, 16)], sem_e
                    ))
                pltpu.make_async_copy(
                    table_hbm.at[idx_v.at[pl.ds(o, CHUNK)]],
                    rows[b], sem_g[b],
                ).wait()
                scat = pltpu.async_copy(
                    rows[b],
                    out_hbm.at[pl.ds(base + o, CHUNK)],
                    sem_s[b],
                )
                for cp in elem_cps:
                    cp.wait()
                for g in range(CHUNK // 16):
                    og = o + g * 16
                    acc = acc + (ls_v[pl.ds(og, 16)] - tl_v[pl.ds(og, 16)])
                scat.wait()
                @pl.when(c + 2 < NCHUNK)
                def _():
                    gather_rows(c + 2, b)
            return acc

        acc = lax.fori_loop(
            0, NCHUNK // 2, pair_body, jnp.zeros((16,), jnp.float32)
        )
        acc_v[...] = acc * (1.0 / BT)
        pltpu.sync_copy(acc_v, part_hbm.at[wid])

    return k(idx_flat, tgt_flat, table_pad, table_flat, lse)


def kernel(index, targets, token_embedding_table):
    idx_flat = jnp.transpose(index).reshape(-1)
    tgt_flat = targets.reshape(-1)
    lse = _row_logsumexp(token_embedding_table).reshape(VOCAB)
    table_pad = jnp.pad(token_embedding_table, ((0, 0), (0, VPAD - VOCAB)))
    logits_pad, part = _sc_gather_and_loss(
        idx_flat, tgt_flat, table_pad,
        token_embedding_table.reshape(-1), lse
    )
    loss = jnp.sum(part)
    return (logits_pad[:, :VOCAB], loss)
